# baseline (speedup 1.0000x reference)
import functools

import jax
import jax.numpy as jnp
import numpy as np
from jax import lax
from jax.experimental import pallas as pl
from jax.experimental.pallas import tpu as pltpu
from jax.experimental.pallas import tpu_sc as plsc

N_NODES = 10000
FEAT = 256
N_EDGES = 160000
ROW_BLOCK = 1000

NC = 2
NS = 16
NW = NC * NS
PB = 320
NPAD = NW * PB
CAP = 12288
CHUNK = 3200
NCHUNK = N_EDGES // CHUNK
EB = 128
FEATP = FEAT // 2
RP = 352
ACCR = PB + 8


def _stage1_body(h_ref, wp_ref, bp_ref, ws_ref, mp_ref, s_ref):
    h = h_ref[...]
    m = jnp.maximum(
        jnp.dot(h, wp_ref[...], preferred_element_type=jnp.float32) + bp_ref[...], 0.0)
    lo = jax.lax.bitcast_convert_type(
        m[:, :FEAT // 2].astype(jnp.bfloat16), jnp.uint16).astype(jnp.uint32)
    hi = jax.lax.bitcast_convert_type(
        m[:, FEAT // 2:].astype(jnp.bfloat16), jnp.uint16).astype(jnp.uint32)
    mp_ref[...] = jax.lax.bitcast_convert_type(lo | (hi << 16), jnp.int32)
    s_ref[...] = jnp.dot(h, ws_ref[...], preferred_element_type=jnp.float32)


def _stage2_body(s_ref, agg_ref, wn_ref, b_ref, o_ref, *, act):
    agg = agg_ref[...].astype(jnp.float32)
    o = (s_ref[...]
         + jnp.dot(agg, wn_ref[...], preferred_element_type=jnp.float32)
         + b_ref[...])
    if act:
        o = jnp.tanh(o + o)
    o_ref[...] = o


def _stage1(h, Wp, bp, Ws):
    n, f = h.shape
    g = Ws.shape[1]
    return pl.pallas_call(
        _stage1_body,
        grid=(n // ROW_BLOCK,),
        in_specs=[
            pl.BlockSpec((ROW_BLOCK, f), lambda i: (i, 0)),
            pl.BlockSpec((f, f), lambda i: (0, 0)),
            pl.BlockSpec((1, f), lambda i: (0, 0)),
            pl.BlockSpec((f, g), lambda i: (0, 0)),
        ],
        out_specs=[
            pl.BlockSpec((ROW_BLOCK, f // 2), lambda i: (i, 0)),
            pl.BlockSpec((ROW_BLOCK, g), lambda i: (i, 0)),
        ],
        out_shape=[
            jax.ShapeDtypeStruct((n, f // 2), jnp.int32),
            jax.ShapeDtypeStruct((n, g), jnp.float32),
        ],
    )(h, Wp, bp.reshape(1, f), Ws)


def _stage2(s, agg, Wn, b, act):
    n, g = s.shape
    f = agg.shape[1]
    return pl.pallas_call(
        functools.partial(_stage2_body, act=act),
        grid=(n // ROW_BLOCK,),
        in_specs=[
            pl.BlockSpec((ROW_BLOCK, g), lambda i: (i, 0)),
            pl.BlockSpec((ROW_BLOCK, f), lambda i: (i, 0)),
            pl.BlockSpec((f, g), lambda i: (0, 0)),
            pl.BlockSpec((1, g), lambda i: (0, 0)),
        ],
        out_specs=pl.BlockSpec((ROW_BLOCK, g), lambda i: (i, 0)),
        out_shape=jax.ShapeDtypeStruct((n, g), jnp.float32),
    )(s, agg, Wn, b.reshape(1, g))


def _sc_mesh():
    return plsc.VectorSubcoreMesh(
        core_axis_name="c", subcore_axis_name="s", num_cores=NC, num_subcores=NS)


_SC_PARAMS = pltpu.CompilerParams(needs_layout_passes=False)


def _worker_id():
    return lax.axis_index("s") * NC + lax.axis_index("c")


def _partition_body(src_hbm, dst_hbm, lsrc_hbm, rptr_hbm, cnt_hbm,
                    src_v, dst_v, lsrc_v, ldl_v, ppos_v, ssrc_v, hist_v, rp_v,
                    cnt_v):
    wid = _worker_id()
    lo = wid * PB
    lo_v = jnp.full((16,), lo, jnp.int32)
    hi_v = lo_v + PB

    def init_body(i, _):
        lsrc_v[pl.ds(i * 16, 16)] = jnp.zeros((16,), jnp.int32)
        ssrc_v[pl.ds(i * 16, 16)] = jnp.zeros((16,), jnp.int32)
        ldl_v[pl.ds(i * 16, 16)] = jnp.full((16,), PB, jnp.int32)
        return 0

    lax.fori_loop(0, CAP // 16, init_body, 0)

    def hzero_body(i, _):
        hist_v[pl.ds(i * 16, 16)] = jnp.zeros((16,), jnp.int32)
        return 0

    lax.fori_loop(0, RP // 16, hzero_body, 0)

    def chunk_body(c, cursor):
        pltpu.sync_copy(src_hbm.at[pl.ds(c * CHUNK, CHUNK)], src_v)
        pltpu.sync_copy(dst_hbm.at[pl.ds(c * CHUNK, CHUNK)], dst_v)

        def vec_body(i, cur):
            d = dst_v[pl.ds(i * 16, 16)]
            s = src_v[pl.ds(i * 16, 16)]
            msk = jnp.logical_and(d >= lo_v, d < hi_v)
            cnt = jnp.sum(jnp.where(msk, 1, 0).astype(jnp.int32))
            plsc.store_compressed(lsrc_v.at[pl.ds(cur, 16)], s, mask=msk)
            plsc.store_compressed(ldl_v.at[pl.ds(cur, 16)], d - lo_v, mask=msk)
            return cur + cnt

        return lax.fori_loop(0, CHUNK // 16, vec_body, cursor)

    total = lax.fori_loop(0, NCHUNK, chunk_body, jnp.int32(0))
    nv = lax.div(total + 15, 16)

    rk0, _ = plsc.scan_count(jnp.zeros((16,), jnp.int32))
    bconv = rk0[0]

    def h_body(i, _):
        dlv = ldl_v[pl.ds(i * 16, 16)]
        rank, lastm = plsc.scan_count(dlv)
        old = plsc.load_gather(hist_v, [dlv])
        plsc.store_scatter(hist_v, [dlv], old + rank + (1 - bconv), mask=lastm)
        ppos_v[pl.ds(i * 16, 16)] = old + rank - bconv
        return 0

    lax.fori_loop(0, nv, h_body, 0)

    def p_body(i, carry):
        v = hist_v[pl.ds(i * 16, 16)]
        c = plsc.cumsum(v)
        rp_v[pl.ds(i * 16, 16)] = c - v + jnp.full((16,), carry, jnp.int32)
        return carry + c[15]

    lax.fori_loop(0, RP // 16, p_body, jnp.int32(0))
    rp_v[pl.ds(PB + 2, 16)] = jnp.full((16,), CAP, jnp.int32)
    rp_v[pl.ds(PB + 16, 16)] = jnp.full((16,), CAP, jnp.int32)

    def s_body(i, _):
        sl = pl.ds(i * 16, 16)
        dlv = ldl_v[sl]
        srcv = lsrc_v[sl]
        pos = plsc.load_gather(rp_v, [dlv]) + ppos_v[sl]
        plsc.store_scatter(ssrc_v, [pos], srcv)
        return 0

    lax.fori_loop(0, nv, s_body, 0)

    cnt_v[...] = jnp.full((16,), total, jnp.int32)
    pltpu.sync_copy(ssrc_v, lsrc_hbm.at[pl.ds(wid * CAP, CAP)])
    pltpu.sync_copy(rp_v, rptr_hbm.at[pl.ds(wid * RP, RP)])
    pltpu.sync_copy(cnt_v, cnt_hbm.at[pl.ds(wid * 16, 16)])


def _sc_partition(src, dst):
    fn = pl.kernel(
        _partition_body,
        out_type=[
            jax.ShapeDtypeStruct((NW * CAP,), jnp.int32),
            jax.ShapeDtypeStruct((NW * RP,), jnp.int32),
            jax.ShapeDtypeStruct((NW * 16,), jnp.int32),
        ],
        mesh=_sc_mesh(),
        compiler_params=_SC_PARAMS,
        scratch_types=[
            pltpu.VMEM((CHUNK,), jnp.int32),
            pltpu.VMEM((CHUNK,), jnp.int32),
            pltpu.VMEM((CAP,), jnp.int32),
            pltpu.VMEM((CAP,), jnp.int32),
            pltpu.VMEM((CAP,), jnp.int32),
            pltpu.VMEM((CAP,), jnp.int32),
            pltpu.VMEM((RP,), jnp.int32),
            pltpu.VMEM((RP,), jnp.int32),
            pltpu.VMEM((16,), jnp.int32),
        ],
    )
    return fn(src, dst)


def _segmax_body(m_hbm, lsrc_hbm, rptr_hbm, cnt_hbm, agg_hbm,
                 idx_v, rows_v, acc_v, rp_v, cnt_vv, sem0, sem1):
    wid = _worker_id()
    base = wid * PB
    sems = (sem0, sem1)
    pltpu.sync_copy(cnt_hbm.at[pl.ds(wid * 16, 16)], cnt_vv)
    pltpu.sync_copy(rptr_hbm.at[pl.ds(wid * RP, RP)], rp_v)
    k_edges = cnt_vv[pl.ds(0, 16)][0]
    nb = lax.div(k_edges + (EB - 1), EB)
    nbu = lax.max(jnp.int32(1), lax.div(nb + 1, 2))

    def rp_at(i):
        return rp_v[pl.ds(i, 16)][0]

    def fire(buf, t):
        off = wid * CAP + lax.min(t * EB, CAP - EB)
        pltpu.sync_copy(lsrc_hbm.at[pl.ds(off, EB)], idx_v.at[buf])
        pltpu.async_copy(m_hbm.at[idx_v.at[buf]], rows_v.at[buf], sems[buf])

    def wait(buf):
        pltpu.make_async_copy(m_hbm.at[idx_v.at[buf]], rows_v.at[buf], sems[buf]).wait()

    zero8 = tuple(jnp.zeros((16,), jnp.int32) for _ in range(FEATP // 16))

    def flushrow(r, regs):
        for k in range(FEATP // 16):
            acc_v[r, pl.ds(k * 16, 16)] = regs[k]

    def process(buf, t, r_in):
        tbase = t * EB

        def row_loop(carry):
            j, r = carry
            gend = rp_at(r + 1)
            e = lax.min(gend - tbase, jnp.int32(EB))

            regs = tuple(
                acc_v[r, pl.ds(k * 16, 16)] for k in range(FEATP // 16))

            def e_body(jj, regs):
                new_regs = []
                for k in range(FEATP // 16):
                    a = plsc.bitcast(regs[k], jnp.bfloat16)
                    rv = plsc.bitcast(rows_v[buf, jj, pl.ds(k * 16, 16)],
                                      jnp.bfloat16)
                    new_regs.append(
                        plsc.bitcast(jnp.maximum(a, rv), jnp.int32))
                return tuple(new_regs)

            def e2_body(p, regs):
                jj = j + 2 * p
                new_regs = []
                for k in range(FEATP // 16):
                    a = plsc.bitcast(regs[k], jnp.bfloat16)
                    r1 = plsc.bitcast(rows_v[buf, jj, pl.ds(k * 16, 16)],
                                      jnp.bfloat16)
                    r2 = plsc.bitcast(rows_v[buf, jj + 1, pl.ds(k * 16, 16)],
                                      jnp.bfloat16)
                    new_regs.append(plsc.bitcast(
                        jnp.maximum(a, jnp.maximum(r1, r2)), jnp.int32))
                return tuple(new_regs)

            npairs = lax.div(e - j, 2)
            regs = lax.fori_loop(0, npairs, e2_body, regs)
            regs = lax.fori_loop(j + 2 * npairs, e, e_body, regs)
            flushrow(r, regs)

            def wcond(c):
                return c[1] <= tbase + e

            def wstep(c):
                rr = c[0] + 1
                return (rr, rp_at(rr + 1))

            r2, _ = lax.while_loop(wcond, wstep, (r, gend))
            return (e, r2)

        def outer_cond(carry):
            return carry[0] < EB

        _, r_out = lax.while_loop(outer_cond, row_loop, (jnp.int32(0), r_in))
        return r_out

    def zero_body(r, _):
        flushrow(r, zero8)
        return 0

    lax.fori_loop(0, ACCR, zero_body, 0)

    fire(0, jnp.int32(0))
    fire(1, jnp.int32(1))

    def u_body(u, r):
        more = u < nbu - 1
        wait(0)
        r = process(0, 2 * u, r)

        @pl.when(more)
        def _():
            fire(0, 2 * u + 2)

        wait(1)
        r = process(1, 2 * u + 1, r)

        @pl.when(more)
        def _():
            fire(1, 2 * u + 3)

        return r

    lax.fori_loop(0, nbu, u_body, jnp.int32(0))
    pltpu.sync_copy(acc_v.at[pl.ds(0, PB)], agg_hbm.at[pl.ds(base, PB)])


def _sc_segmax(m, lsrc, rptr, cnt):
    fn = pl.kernel(
        _segmax_body,
        out_type=jax.ShapeDtypeStruct((NPAD, FEATP), jnp.int32),
        mesh=_sc_mesh(),
        compiler_params=_SC_PARAMS,
        scratch_types=[
            pltpu.VMEM((2, EB), jnp.int32),
            pltpu.VMEM((2, EB, FEATP), jnp.int32),
            pltpu.VMEM((ACCR, FEATP), jnp.int32),
            pltpu.VMEM((RP,), jnp.int32),
            pltpu.VMEM((16,), jnp.int32),
            pltpu.SemaphoreType.DMA,
            pltpu.SemaphoreType.DMA,
        ],
    )
    agg_pack = fn(m, lsrc, rptr, cnt)[:N_NODES]
    pairs = jax.lax.bitcast_convert_type(agg_pack, jnp.uint16)
    agg = jax.lax.bitcast_convert_type(pairs, jnp.bfloat16).reshape(N_NODES, FEAT)
    return agg


_PERM = np.empty((FEAT,), dtype=np.int32)
_PERM[0::2] = np.arange(FEAT // 2)
_PERM[1::2] = np.arange(FEAT // 2, FEAT)


def _layer(h, lists, Wp, bp, Ws, Wn, b, act):
    mp, s = _stage1(h, Wp, bp, Ws)
    agg = _sc_segmax(mp, *lists)
    return _stage2(s, agg, Wn[_PERM], b, act)


def kernel(inputs, edge_index, Wp1, bp1, Ws1, Wn1, b1, Wp2, bp2, Ws2, Wn2, b2, Wp3, bp3, Ws3, Wn3, b3):
    src = edge_index[0]
    dst = edge_index[1]
    lists = _sc_partition(src, dst)
    h = _layer(inputs, lists, Wp1, bp1, Ws1, Wn1, b1, act=True)
    h = _layer(h, lists, Wp2, bp2, Ws2, Wn2, b2, act=True)
    h = _layer(h, lists, Wp2, bp2, Ws2, Wn2, b2, act=True)
    h = _layer(h, lists, Wp3, bp3, Ws3, Wn3, b3, act=False)
    return h


# speedup vs baseline: 1.0272x; 1.0272x over previous
import functools

import jax
import jax.numpy as jnp
import numpy as np
from jax import lax
from jax.experimental import pallas as pl
from jax.experimental.pallas import tpu as pltpu
from jax.experimental.pallas import tpu_sc as plsc

N_NODES = 10000
FEAT = 256
N_EDGES = 160000
ROW_BLOCK = 1000

NC = 2
NS = 16
NW = NC * NS
PB = 320
NPAD = NW * PB
CAP = 12288
CHUNK = 3200
NCHUNK = N_EDGES // CHUNK
EB = 128
FEATP = FEAT // 2
RP = 352
ACCR = PB + 8


def _stage1_body(h_ref, wp_ref, bp_ref, ws_ref, mp_ref, s_ref):
    h = h_ref[...]
    m = jnp.maximum(
        jnp.dot(h, wp_ref[...], preferred_element_type=jnp.float32) + bp_ref[...], 0.0)
    lo = jax.lax.bitcast_convert_type(
        m[:, :FEAT // 2].astype(jnp.bfloat16), jnp.uint16).astype(jnp.uint32)
    hi = jax.lax.bitcast_convert_type(
        m[:, FEAT // 2:].astype(jnp.bfloat16), jnp.uint16).astype(jnp.uint32)
    mp_ref[...] = jax.lax.bitcast_convert_type(lo | (hi << 16), jnp.int32)
    s_ref[...] = jnp.dot(h, ws_ref[...], preferred_element_type=jnp.float32)


def _stage2_body(s_ref, agg_ref, wn_ref, b_ref, o_ref, *, act):
    agg = agg_ref[...].astype(jnp.float32)
    o = (s_ref[...]
         + jnp.dot(agg, wn_ref[...], preferred_element_type=jnp.float32)
         + b_ref[...])
    if act:
        o = jnp.tanh(o + o)
    o_ref[...] = o


def _stage1(h, Wp, bp, Ws):
    n, f = h.shape
    g = Ws.shape[1]
    return pl.pallas_call(
        _stage1_body,
        grid=(n // ROW_BLOCK,),
        in_specs=[
            pl.BlockSpec((ROW_BLOCK, f), lambda i: (i, 0)),
            pl.BlockSpec((f, f), lambda i: (0, 0)),
            pl.BlockSpec((1, f), lambda i: (0, 0)),
            pl.BlockSpec((f, g), lambda i: (0, 0)),
        ],
        out_specs=[
            pl.BlockSpec((ROW_BLOCK, f // 2), lambda i: (i, 0)),
            pl.BlockSpec((ROW_BLOCK, g), lambda i: (i, 0)),
        ],
        out_shape=[
            jax.ShapeDtypeStruct((n, f // 2), jnp.int32),
            jax.ShapeDtypeStruct((n, g), jnp.float32),
        ],
    )(h, Wp, bp.reshape(1, f), Ws)


def _stage2(s, agg, Wn, b, act):
    n, g = s.shape
    f = agg.shape[1]
    return pl.pallas_call(
        functools.partial(_stage2_body, act=act),
        grid=(n // ROW_BLOCK,),
        in_specs=[
            pl.BlockSpec((ROW_BLOCK, g), lambda i: (i, 0)),
            pl.BlockSpec((ROW_BLOCK, f), lambda i: (i, 0)),
            pl.BlockSpec((f, g), lambda i: (0, 0)),
            pl.BlockSpec((1, g), lambda i: (0, 0)),
        ],
        out_specs=pl.BlockSpec((ROW_BLOCK, g), lambda i: (i, 0)),
        out_shape=jax.ShapeDtypeStruct((n, g), jnp.float32),
    )(s, agg, Wn, b.reshape(1, g))


def _fused_body(s_ref, agg_ref, wn_ref, b_ref, wp_ref, bp_ref, ws_ref,
                mp_ref, s2_ref):
    agg = agg_ref[...].astype(jnp.float32)
    o = (s_ref[...]
         + jnp.dot(agg, wn_ref[...], preferred_element_type=jnp.float32)
         + b_ref[...])
    h = jnp.tanh(o + o)
    m = jnp.maximum(
        jnp.dot(h, wp_ref[...], preferred_element_type=jnp.float32) + bp_ref[...], 0.0)
    lo = jax.lax.bitcast_convert_type(
        m[:, :FEAT // 2].astype(jnp.bfloat16), jnp.uint16).astype(jnp.uint32)
    hi = jax.lax.bitcast_convert_type(
        m[:, FEAT // 2:].astype(jnp.bfloat16), jnp.uint16).astype(jnp.uint32)
    mp_ref[...] = jax.lax.bitcast_convert_type(lo | (hi << 16), jnp.int32)
    s2_ref[...] = jnp.dot(h, ws_ref[...], preferred_element_type=jnp.float32)


def _fused_stage(s, agg, Wn, b, Wp, bp, Ws):
    n, f = s.shape
    g = Ws.shape[1]
    return pl.pallas_call(
        _fused_body,
        grid=(n // ROW_BLOCK,),
        in_specs=[
            pl.BlockSpec((ROW_BLOCK, f), lambda i: (i, 0)),
            pl.BlockSpec((ROW_BLOCK, f), lambda i: (i, 0)),
            pl.BlockSpec((f, f), lambda i: (0, 0)),
            pl.BlockSpec((1, f), lambda i: (0, 0)),
            pl.BlockSpec((f, f), lambda i: (0, 0)),
            pl.BlockSpec((1, f), lambda i: (0, 0)),
            pl.BlockSpec((f, g), lambda i: (0, 0)),
        ],
        out_specs=[
            pl.BlockSpec((ROW_BLOCK, f // 2), lambda i: (i, 0)),
            pl.BlockSpec((ROW_BLOCK, g), lambda i: (i, 0)),
        ],
        out_shape=[
            jax.ShapeDtypeStruct((n, f // 2), jnp.int32),
            jax.ShapeDtypeStruct((n, g), jnp.float32),
        ],
    )(s, agg, Wn[_PERM], b.reshape(1, f), Wp, bp.reshape(1, f), Ws)


def _sc_mesh():
    return plsc.VectorSubcoreMesh(
        core_axis_name="c", subcore_axis_name="s", num_cores=NC, num_subcores=NS)


_SC_PARAMS = pltpu.CompilerParams(needs_layout_passes=False)


def _worker_id():
    return lax.axis_index("s") * NC + lax.axis_index("c")


def _partition_body(src_hbm, dst_hbm, lsrc_hbm, rptr_hbm, cnt_hbm,
                    src_v, dst_v, lsrc_v, ldl_v, ppos_v, ssrc_v, hist_v, rp_v,
                    cnt_v):
    wid = _worker_id()
    lo = wid * PB
    lo_v = jnp.full((16,), lo, jnp.int32)
    hi_v = lo_v + PB

    def init_body(i, _):
        lsrc_v[pl.ds(i * 16, 16)] = jnp.zeros((16,), jnp.int32)
        ssrc_v[pl.ds(i * 16, 16)] = jnp.zeros((16,), jnp.int32)
        ldl_v[pl.ds(i * 16, 16)] = jnp.full((16,), PB, jnp.int32)
        return 0

    lax.fori_loop(0, CAP // 16, init_body, 0)

    def hzero_body(i, _):
        hist_v[pl.ds(i * 16, 16)] = jnp.zeros((16,), jnp.int32)
        return 0

    lax.fori_loop(0, RP // 16, hzero_body, 0)

    def chunk_body(c, cursor):
        pltpu.sync_copy(src_hbm.at[pl.ds(c * CHUNK, CHUNK)], src_v)
        pltpu.sync_copy(dst_hbm.at[pl.ds(c * CHUNK, CHUNK)], dst_v)

        def vec_body(i, cur):
            d = dst_v[pl.ds(i * 16, 16)]
            s = src_v[pl.ds(i * 16, 16)]
            msk = jnp.logical_and(d >= lo_v, d < hi_v)
            cnt = jnp.sum(jnp.where(msk, 1, 0).astype(jnp.int32))
            plsc.store_compressed(lsrc_v.at[pl.ds(cur, 16)], s, mask=msk)
            plsc.store_compressed(ldl_v.at[pl.ds(cur, 16)], d - lo_v, mask=msk)
            return cur + cnt

        return lax.fori_loop(0, CHUNK // 16, vec_body, cursor)

    total = lax.fori_loop(0, NCHUNK, chunk_body, jnp.int32(0))
    nv = lax.div(total + 15, 16)

    rk0, _ = plsc.scan_count(jnp.zeros((16,), jnp.int32))
    bconv = rk0[0]

    def h_body(i, _):
        dlv = ldl_v[pl.ds(i * 16, 16)]
        rank, lastm = plsc.scan_count(dlv)
        old = plsc.load_gather(hist_v, [dlv])
        plsc.store_scatter(hist_v, [dlv], old + rank + (1 - bconv), mask=lastm)
        ppos_v[pl.ds(i * 16, 16)] = old + rank - bconv
        return 0

    lax.fori_loop(0, nv, h_body, 0)

    def p_body(i, carry):
        v = hist_v[pl.ds(i * 16, 16)]
        c = plsc.cumsum(v)
        rp_v[pl.ds(i * 16, 16)] = c - v + jnp.full((16,), carry, jnp.int32)
        return carry + c[15]

    lax.fori_loop(0, RP // 16, p_body, jnp.int32(0))
    rp_v[pl.ds(PB + 2, 16)] = jnp.full((16,), CAP, jnp.int32)
    rp_v[pl.ds(PB + 16, 16)] = jnp.full((16,), CAP, jnp.int32)

    def s_body(i, _):
        sl = pl.ds(i * 16, 16)
        dlv = ldl_v[sl]
        srcv = lsrc_v[sl]
        pos = plsc.load_gather(rp_v, [dlv]) + ppos_v[sl]
        plsc.store_scatter(ssrc_v, [pos], srcv)
        return 0

    lax.fori_loop(0, nv, s_body, 0)

    cnt_v[...] = jnp.full((16,), total, jnp.int32)
    pltpu.sync_copy(ssrc_v, lsrc_hbm.at[pl.ds(wid * CAP, CAP)])
    pltpu.sync_copy(rp_v, rptr_hbm.at[pl.ds(wid * RP, RP)])
    pltpu.sync_copy(cnt_v, cnt_hbm.at[pl.ds(wid * 16, 16)])


def _sc_partition(src, dst):
    fn = pl.kernel(
        _partition_body,
        out_type=[
            jax.ShapeDtypeStruct((NW * CAP,), jnp.int32),
            jax.ShapeDtypeStruct((NW * RP,), jnp.int32),
            jax.ShapeDtypeStruct((NW * 16,), jnp.int32),
        ],
        mesh=_sc_mesh(),
        compiler_params=_SC_PARAMS,
        scratch_types=[
            pltpu.VMEM((CHUNK,), jnp.int32),
            pltpu.VMEM((CHUNK,), jnp.int32),
            pltpu.VMEM((CAP,), jnp.int32),
            pltpu.VMEM((CAP,), jnp.int32),
            pltpu.VMEM((CAP,), jnp.int32),
            pltpu.VMEM((CAP,), jnp.int32),
            pltpu.VMEM((RP,), jnp.int32),
            pltpu.VMEM((RP,), jnp.int32),
            pltpu.VMEM((16,), jnp.int32),
        ],
    )
    return fn(src, dst)


def _segmax_body(m_hbm, lsrc_hbm, rptr_hbm, cnt_hbm, agg_hbm,
                 idx_v, rows_v, acc_v, rp_v, cnt_vv, sem0, sem1):
    wid = _worker_id()
    base = wid * PB
    sems = (sem0, sem1)
    pltpu.sync_copy(cnt_hbm.at[pl.ds(wid * 16, 16)], cnt_vv)
    pltpu.sync_copy(rptr_hbm.at[pl.ds(wid * RP, RP)], rp_v)
    k_edges = cnt_vv[pl.ds(0, 16)][0]
    nb = lax.div(k_edges + (EB - 1), EB)
    nbu = lax.max(jnp.int32(1), lax.div(nb + 1, 2))

    def rp_at(i):
        return rp_v[pl.ds(i, 16)][0]

    def fire(buf, t):
        off = wid * CAP + lax.min(t * EB, CAP - EB)
        pltpu.sync_copy(lsrc_hbm.at[pl.ds(off, EB)], idx_v.at[buf])
        pltpu.async_copy(m_hbm.at[idx_v.at[buf]], rows_v.at[buf], sems[buf])

    def wait(buf):
        pltpu.make_async_copy(m_hbm.at[idx_v.at[buf]], rows_v.at[buf], sems[buf]).wait()

    zero8 = tuple(jnp.zeros((16,), jnp.int32) for _ in range(FEATP // 16))

    def flushrow(r, regs):
        for k in range(FEATP // 16):
            acc_v[r, pl.ds(k * 16, 16)] = regs[k]

    def process(buf, t, r_in):
        tbase = t * EB

        def row_loop(carry):
            j, r = carry
            gend = rp_at(r + 1)
            e = lax.min(gend - tbase, jnp.int32(EB))

            regs = tuple(
                acc_v[r, pl.ds(k * 16, 16)] for k in range(FEATP // 16))

            def e_body(jj, regs):
                new_regs = []
                for k in range(FEATP // 16):
                    a = plsc.bitcast(regs[k], jnp.bfloat16)
                    rv = plsc.bitcast(rows_v[buf, jj, pl.ds(k * 16, 16)],
                                      jnp.bfloat16)
                    new_regs.append(
                        plsc.bitcast(jnp.maximum(a, rv), jnp.int32))
                return tuple(new_regs)

            regs = lax.fori_loop(j, e, e_body, regs)
            flushrow(r, regs)

            def wcond(c):
                return c[1] <= tbase + e

            def wstep(c):
                rr = c[0] + 1
                return (rr, rp_at(rr + 1))

            r2, _ = lax.while_loop(wcond, wstep, (r, gend))
            return (e, r2)

        def outer_cond(carry):
            return carry[0] < EB

        _, r_out = lax.while_loop(outer_cond, row_loop, (jnp.int32(0), r_in))
        return r_out

    def zero_body(r, _):
        flushrow(r, zero8)
        return 0

    lax.fori_loop(0, ACCR, zero_body, 0)

    fire(0, jnp.int32(0))
    fire(1, jnp.int32(1))

    def u_body(u, r):
        more = u < nbu - 1
        wait(0)
        r = process(0, 2 * u, r)

        @pl.when(more)
        def _():
            fire(0, 2 * u + 2)

        wait(1)
        r = process(1, 2 * u + 1, r)

        @pl.when(more)
        def _():
            fire(1, 2 * u + 3)

        return r

    lax.fori_loop(0, nbu, u_body, jnp.int32(0))
    pltpu.sync_copy(acc_v.at[pl.ds(0, PB)], agg_hbm.at[pl.ds(base, PB)])


def _sc_segmax(m, lsrc, rptr, cnt):
    fn = pl.kernel(
        _segmax_body,
        out_type=jax.ShapeDtypeStruct((NPAD, FEATP), jnp.int32),
        mesh=_sc_mesh(),
        compiler_params=_SC_PARAMS,
        scratch_types=[
            pltpu.VMEM((2, EB), jnp.int32),
            pltpu.VMEM((2, EB, FEATP), jnp.int32),
            pltpu.VMEM((ACCR, FEATP), jnp.int32),
            pltpu.VMEM((RP,), jnp.int32),
            pltpu.VMEM((16,), jnp.int32),
            pltpu.SemaphoreType.DMA,
            pltpu.SemaphoreType.DMA,
        ],
    )
    agg_pack = fn(m, lsrc, rptr, cnt)[:N_NODES]
    pairs = jax.lax.bitcast_convert_type(agg_pack, jnp.uint16)
    agg = jax.lax.bitcast_convert_type(pairs, jnp.bfloat16).reshape(N_NODES, FEAT)
    return agg


_PERM = np.empty((FEAT,), dtype=np.int32)
_PERM[0::2] = np.arange(FEAT // 2)
_PERM[1::2] = np.arange(FEAT // 2, FEAT)


def kernel(inputs, edge_index, Wp1, bp1, Ws1, Wn1, b1, Wp2, bp2, Ws2, Wn2, b2, Wp3, bp3, Ws3, Wn3, b3):
    src = edge_index[0]
    dst = edge_index[1]
    lists = _sc_partition(src, dst)
    mp, s = _stage1(inputs, Wp1, bp1, Ws1)
    agg = _sc_segmax(mp, *lists)
    mp, s = _fused_stage(s, agg, Wn1, b1, Wp2, bp2, Ws2)
    agg = _sc_segmax(mp, *lists)
    mp, s = _fused_stage(s, agg, Wn2, b2, Wp2, bp2, Ws2)
    agg = _sc_segmax(mp, *lists)
    mp, s = _fused_stage(s, agg, Wn2, b2, Wp3, bp3, Ws3)
    agg = _sc_segmax(mp, *lists)
    return _stage2(s, agg, Wn3[_PERM], b3, act=False)
